# initial kernel scaffold (unmeasured)
import jax
import jax.numpy as jnp
from jax import lax
from jax.experimental import pallas as pl
from jax.experimental.pallas import tpu as pltpu


def kernel(
    x,
):
    def body(*refs):
        pass

    out_shape = jax.ShapeDtypeStruct(..., jnp.float32)
    return pl.pallas_call(body, out_shape=out_shape)(...)


# baseline (device time: 22598 ns/iter reference)
import jax
import jax.numpy as jnp
from jax import lax
from jax.experimental import pallas as pl
from jax.experimental.pallas import tpu as pltpu

N_DEV = 4


def kernel(x):
    m, n = x.shape

    def body(
        x_ref,
        out_ref,
        top_halo,
        bot_halo,
        send_sem_dn,
        send_sem_up,
        recv_sem_top,
        recv_sem_bot,
    ):
        my = lax.axis_index("i")
        up = my - 1
        dn = my + 1

        @pl.when(my < N_DEV - 1)
        def _():
            rdma = pltpu.make_async_remote_copy(
                src_ref=x_ref.at[pl.ds(m - 1, 1), :],
                dst_ref=top_halo,
                send_sem=send_sem_dn,
                recv_sem=recv_sem_top,
                device_id=(dn,),
                device_id_type=pl.DeviceIdType.MESH,
            )
            rdma.start()
            rdma.wait_send()

        @pl.when(my > 0)
        def _():
            rdma = pltpu.make_async_remote_copy(
                src_ref=x_ref.at[pl.ds(0, 1), :],
                dst_ref=bot_halo,
                send_sem=send_sem_up,
                recv_sem=recv_sem_bot,
                device_id=(up,),
                device_id_type=pl.DeviceIdType.MESH,
            )
            rdma.start()
            rdma.wait_send()

        out_ref[pl.ds(1, m - 2), :] = (
            0.25 * x_ref[pl.ds(0, m - 2), :]
            + 0.5 * x_ref[pl.ds(1, m - 2), :]
            + 0.25 * x_ref[pl.ds(2, m - 2), :]
        )

        @pl.when(my == 0)
        def _():
            out_ref[pl.ds(0, 1), :] = x_ref[pl.ds(0, 1), :]

        @pl.when(my > 0)
        def _():
            recv = pltpu.make_async_remote_copy(
                src_ref=top_halo,
                dst_ref=top_halo,
                send_sem=send_sem_dn,
                recv_sem=recv_sem_top,
                device_id=(up,),
                device_id_type=pl.DeviceIdType.MESH,
            )
            recv.wait_recv()
            out_ref[pl.ds(0, 1), :] = (
                0.25 * top_halo[:, :]
                + 0.5 * x_ref[pl.ds(0, 1), :]
                + 0.25 * x_ref[pl.ds(1, 1), :]
            )

        @pl.when(my == N_DEV - 1)
        def _():
            out_ref[pl.ds(m - 1, 1), :] = x_ref[pl.ds(m - 1, 1), :]

        @pl.when(my < N_DEV - 1)
        def _():
            recv = pltpu.make_async_remote_copy(
                src_ref=bot_halo,
                dst_ref=bot_halo,
                send_sem=send_sem_up,
                recv_sem=recv_sem_bot,
                device_id=(dn,),
                device_id_type=pl.DeviceIdType.MESH,
            )
            recv.wait_recv()
            out_ref[pl.ds(m - 1, 1), :] = (
                0.25 * x_ref[pl.ds(m - 2, 1), :]
                + 0.5 * x_ref[pl.ds(m - 1, 1), :]
                + 0.25 * bot_halo[:, :]
            )

    return pl.pallas_call(
        body,
        out_shape=jax.ShapeDtypeStruct((m, n), x.dtype),
        in_specs=[pl.BlockSpec(memory_space=pltpu.VMEM)],
        out_specs=pl.BlockSpec(memory_space=pltpu.VMEM),
        scratch_shapes=[
            pltpu.VMEM((1, n), x.dtype),
            pltpu.VMEM((1, n), x.dtype),
            pltpu.SemaphoreType.DMA,
            pltpu.SemaphoreType.DMA,
            pltpu.SemaphoreType.DMA,
            pltpu.SemaphoreType.DMA,
        ],
    )(x)


# device time: 21943 ns/iter; 1.0299x vs baseline; 1.0299x over previous
import jax
import jax.numpy as jnp
from jax import lax
from jax.experimental import pallas as pl
from jax.experimental.pallas import tpu as pltpu

N_DEV = 4
K = 8
P = 8


def kernel(x):
    m, n = x.shape
    C = m // K

    def body(
        x_ref,
        out_ref,
        inbuf,
        outbuf,
        in_sems,
        out_sems,
        top_halo,
        bot_halo,
        send_sem_dn,
        send_sem_up,
        recv_sem_top,
        recv_sem_bot,
    ):
        my = lax.axis_index("i")
        up = my - 1
        dn = my + 1

        send_dn = pltpu.make_async_remote_copy(
            src_ref=x_ref.at[pl.ds(m - P, P), :],
            dst_ref=top_halo,
            send_sem=send_sem_dn,
            recv_sem=recv_sem_top,
            device_id=(dn,),
            device_id_type=pl.DeviceIdType.MESH,
        )
        send_up = pltpu.make_async_remote_copy(
            src_ref=x_ref.at[pl.ds(0, P), :],
            dst_ref=bot_halo,
            send_sem=send_sem_up,
            recv_sem=recv_sem_bot,
            device_id=(up,),
            device_id_type=pl.DeviceIdType.MESH,
        )

        @pl.when(my < N_DEV - 1)
        def _():
            send_dn.start()

        @pl.when(my > 0)
        def _():
            send_up.start()

        def in_copy(k, slot):
            if k == 0:
                return pltpu.make_async_copy(
                    x_ref.at[pl.ds(0, C + P), :],
                    inbuf.at[slot, pl.ds(P, C + P), :],
                    in_sems.at[slot],
                )
            if k == K - 1:
                return pltpu.make_async_copy(
                    x_ref.at[pl.ds(k * C - P, C + P), :],
                    inbuf.at[slot, pl.ds(0, C + P), :],
                    in_sems.at[slot],
                )
            return pltpu.make_async_copy(
                x_ref.at[pl.ds(k * C - P, C + 2 * P), :],
                inbuf.at[slot, :, :],
                in_sems.at[slot],
            )

        def out_copy(k, slot):
            return pltpu.make_async_copy(
                outbuf.at[slot],
                out_ref.at[pl.ds(k * C, C), :],
                out_sems.at[slot],
            )

        order = list(range(1, K - 1)) + [0, K - 1]

        copies_in = {}
        copies_out = {}

        copies_in[order[0]] = in_copy(order[0], 0)
        copies_in[order[0]].start()

        for idx, k in enumerate(order):
            slot = idx % 2
            if idx + 1 < K:
                nk = order[idx + 1]
                copies_in[nk] = in_copy(nk, (idx + 1) % 2)
                copies_in[nk].start()
            copies_in[k].wait()

            if k == 0:
                @pl.when(my > 0)
                def _():
                    recv = pltpu.make_async_remote_copy(
                        src_ref=top_halo,
                        dst_ref=top_halo,
                        send_sem=send_sem_dn,
                        recv_sem=recv_sem_top,
                        device_id=(up,),
                        device_id_type=pl.DeviceIdType.MESH,
                    )
                    recv.wait_recv()
                    inbuf[slot, pl.ds(0, P), :] = top_halo[:, :]
            if k == K - 1:
                @pl.when(my < N_DEV - 1)
                def _():
                    recv = pltpu.make_async_remote_copy(
                        src_ref=bot_halo,
                        dst_ref=bot_halo,
                        send_sem=send_sem_up,
                        recv_sem=recv_sem_bot,
                        device_id=(dn,),
                        device_id_type=pl.DeviceIdType.MESH,
                    )
                    recv.wait_recv()
                    inbuf[slot, pl.ds(C + P, P), :] = bot_halo[:, :]

            if idx >= 2:
                copies_out[order[idx - 2]].wait()

            outbuf[slot, :, :] = (
                0.25 * inbuf[slot, pl.ds(P - 1, C), :]
                + 0.5 * inbuf[slot, pl.ds(P, C), :]
                + 0.25 * inbuf[slot, pl.ds(P + 1, C), :]
            )

            if k == 0:
                @pl.when(my == 0)
                def _():
                    outbuf[slot, pl.ds(0, 1), :] = inbuf[slot, pl.ds(P, 1), :]
            if k == K - 1:
                @pl.when(my == N_DEV - 1)
                def _():
                    outbuf[slot, pl.ds(C - 1, 1), :] = (
                        inbuf[slot, pl.ds(C + P - 1, 1), :]
                    )

            copies_out[k] = out_copy(k, slot)
            copies_out[k].start()

        copies_out[order[K - 2]].wait()
        copies_out[order[K - 1]].wait()

        @pl.when(my < N_DEV - 1)
        def _():
            send_dn.wait_send()

        @pl.when(my > 0)
        def _():
            send_up.wait_send()

    return pl.pallas_call(
        body,
        out_shape=jax.ShapeDtypeStruct((m, n), x.dtype),
        in_specs=[pl.BlockSpec(memory_space=pl.ANY)],
        out_specs=pl.BlockSpec(memory_space=pl.ANY),
        scratch_shapes=[
            pltpu.VMEM((2, C + 2 * P, n), x.dtype),
            pltpu.VMEM((2, C, n), x.dtype),
            pltpu.SemaphoreType.DMA((2,)),
            pltpu.SemaphoreType.DMA((2,)),
            pltpu.VMEM((P, n), x.dtype),
            pltpu.VMEM((P, n), x.dtype),
            pltpu.SemaphoreType.DMA,
            pltpu.SemaphoreType.DMA,
            pltpu.SemaphoreType.DMA,
            pltpu.SemaphoreType.DMA,
        ],
    )(x)


# device time: 14688 ns/iter; 1.5385x vs baseline; 1.4939x over previous
import jax
import jax.numpy as jnp
from jax import lax
from jax.experimental import pallas as pl
from jax.experimental.pallas import tpu as pltpu

N_DEV = 4
K = 4
S = 3
P = 8


def kernel(x):
    m, n = x.shape
    C = m // K
    B = C + 2 * P

    def body(
        x_ref,
        out_ref,
        inbuf,
        outbuf,
        in_sems,
        out_sems,
        top_halo,
        bot_halo,
        send_sem_dn,
        send_sem_up,
        recv_sem_top,
        recv_sem_bot,
    ):
        my = lax.axis_index("i")
        up = my - 1
        dn = my + 1

        barrier_sem = pltpu.get_barrier_semaphore()

        @pl.when(my > 0)
        def _():
            pl.semaphore_signal(
                barrier_sem, inc=1,
                device_id=(up,), device_id_type=pl.DeviceIdType.MESH,
            )

        @pl.when(my < N_DEV - 1)
        def _():
            pl.semaphore_signal(
                barrier_sem, inc=1,
                device_id=(dn,), device_id_type=pl.DeviceIdType.MESH,
            )

        send_dn = pltpu.make_async_remote_copy(
            src_ref=x_ref.at[pl.ds(m - P, P), :],
            dst_ref=top_halo,
            send_sem=send_sem_dn,
            recv_sem=recv_sem_top,
            device_id=(dn,),
            device_id_type=pl.DeviceIdType.MESH,
        )
        send_up = pltpu.make_async_remote_copy(
            src_ref=x_ref.at[pl.ds(0, P), :],
            dst_ref=bot_halo,
            send_sem=send_sem_up,
            recv_sem=recv_sem_bot,
            device_id=(up,),
            device_id_type=pl.DeviceIdType.MESH,
        )

        def in_copy(k, slot):
            if k == 0:
                return pltpu.make_async_copy(
                    x_ref.at[pl.ds(0, C + P), :],
                    inbuf.at[slot, pl.ds(P, C + P), :],
                    in_sems.at[slot],
                )
            if k == K - 1:
                return pltpu.make_async_copy(
                    x_ref.at[pl.ds(k * C - P, C + P), :],
                    inbuf.at[slot, pl.ds(0, C + P), :],
                    in_sems.at[slot],
                )
            return pltpu.make_async_copy(
                x_ref.at[pl.ds(k * C - P, B), :],
                inbuf.at[slot, :, :],
                in_sems.at[slot],
            )

        def out_copy(k, slot):
            return pltpu.make_async_copy(
                outbuf.at[slot],
                out_ref.at[pl.ds(k * C, C), :],
                out_sems.at[slot],
            )

        order = list(range(1, K - 1)) + [0, K - 1]

        copies_in = {}
        copies_out = {}

        for j in range(S - 1):
            copies_in[order[j]] = in_copy(order[j], j % S)
            copies_in[order[j]].start()

        is_edge = jnp.logical_or(my == 0, my == N_DEV - 1)

        @pl.when(is_edge)
        def _():
            pl.semaphore_wait(barrier_sem, 1)

        @pl.when(jnp.logical_not(is_edge))
        def _():
            pl.semaphore_wait(barrier_sem, 2)


        @pl.when(my < N_DEV - 1)
        def _():
            send_dn.start()

        @pl.when(my > 0)
        def _():
            send_up.start()


        for idx, k in enumerate(order):
            slot = idx % S
            if idx + S - 1 < K:
                nk = order[idx + S - 1]
                copies_in[nk] = in_copy(nk, (idx + S - 1) % S)
                copies_in[nk].start()
            copies_in[k].wait()

            if k == 0:
                @pl.when(my > 0)
                def _():
                    recv = pltpu.make_async_remote_copy(
                        src_ref=top_halo,
                        dst_ref=top_halo,
                        send_sem=send_sem_dn,
                        recv_sem=recv_sem_top,
                        device_id=(up,),
                        device_id_type=pl.DeviceIdType.MESH,
                    )
                    recv.wait_recv()
                    inbuf[slot, pl.ds(0, P), :] = top_halo[:, :]
            if k == K - 1:
                @pl.when(my < N_DEV - 1)
                def _():
                    recv = pltpu.make_async_remote_copy(
                        src_ref=bot_halo,
                        dst_ref=bot_halo,
                        send_sem=send_sem_up,
                        recv_sem=recv_sem_bot,
                        device_id=(dn,),
                        device_id_type=pl.DeviceIdType.MESH,
                    )
                    recv.wait_recv()
                    inbuf[slot, pl.ds(C + P, P), :] = bot_halo[:, :]

            if idx >= S:
                copies_out[order[idx - S]].wait()

            a = inbuf[slot, :, :]
            am1 = pltpu.roll(a, 1, 0)
            ap1 = pltpu.roll(a, B - 1, 0)
            r = 0.25 * am1 + 0.5 * a + 0.25 * ap1
            outbuf[slot, :, :] = r[P:P + C, :]

            if k == 0:
                @pl.when(my == 0)
                def _():
                    outbuf[slot, pl.ds(0, 1), :] = inbuf[slot, pl.ds(P, 1), :]
            if k == K - 1:
                @pl.when(my == N_DEV - 1)
                def _():
                    outbuf[slot, pl.ds(C - 1, 1), :] = (
                        inbuf[slot, pl.ds(C + P - 1, 1), :]
                    )

            copies_out[k] = out_copy(k, slot)
            copies_out[k].start()

        for j in range(max(0, K - S), K):
            copies_out[order[j]].wait()

        @pl.when(my < N_DEV - 1)
        def _():
            send_dn.wait_send()

        @pl.when(my > 0)
        def _():
            send_up.wait_send()

    return pl.pallas_call(
        body,
        out_shape=jax.ShapeDtypeStruct((m, n), x.dtype),
        in_specs=[pl.BlockSpec(memory_space=pl.ANY)],
        out_specs=pl.BlockSpec(memory_space=pl.ANY),
        scratch_shapes=[
            pltpu.VMEM((S, C + 2 * P, n), x.dtype),
            pltpu.VMEM((S, C, n), x.dtype),
            pltpu.SemaphoreType.DMA((S,)),
            pltpu.SemaphoreType.DMA((S,)),
            pltpu.VMEM((P, n), x.dtype),
            pltpu.VMEM((P, n), x.dtype),
            pltpu.SemaphoreType.DMA,
            pltpu.SemaphoreType.DMA,
            pltpu.SemaphoreType.DMA,
            pltpu.SemaphoreType.DMA,
        ],
        compiler_params=pltpu.CompilerParams(collective_id=0),
    )(x)
